# baseline (speedup 1.0000x reference)
import jax
import jax.numpy as jnp
from jax.experimental import pallas as pl
from jax.experimental.pallas import tpu as pltpu

B, L, E = 1, 2048, 768
N_EXPERTS = 8
TOP_K = 2
MLP_DIM = 2048

CAP = L
ROW_BLK = 128
BLKS_PER_E = CAP // ROW_BLK


def _moe_kernel(x_ref, gate_ref, w0_ref, w1_ref, wo_ref, out_ref,
                d1_ref, d2_ref, p1_ref, p2_ref, cnt_ref):
    n = pl.program_id(0)

    @pl.when(n == 0)
    def _route():
        logits = jnp.dot(x_ref[...], gate_ref[...],
                         preferred_element_type=jnp.float32)
        lane = jax.lax.broadcasted_iota(jnp.int32, (L, N_EXPERTS), 1)
        a1 = jnp.argmax(logits, axis=-1)[:, None]
        m1 = jnp.max(logits, axis=-1, keepdims=True)
        masked = jnp.where(lane == a1, -jnp.inf, logits)
        a2 = jnp.argmax(masked, axis=-1)[:, None]
        m2 = jnp.max(masked, axis=-1, keepdims=True)
        e2 = jnp.exp(m2 - m1)
        p1_ref[...] = 1.0 / (1.0 + e2)
        p2_ref[...] = e2 / (1.0 + e2)

        a_mat = ((lane == a1) | (lane == a2)).astype(jnp.float32)
        a_b = a_mat.astype(jnp.bfloat16)
        chunks = []
        for c in range(4):
            row = (jax.lax.broadcasted_iota(jnp.int32, (L // 4, L), 0)
                   + c * (L // 4))
            col = jax.lax.broadcasted_iota(jnp.int32, (L // 4, L), 1)
            lower_c = (row > col).astype(jnp.bfloat16)
            chunks.append(jnp.dot(lower_c, a_b,
                                  preferred_element_type=jnp.float32))
        rank = jnp.concatenate(chunks, axis=0)
        csum = rank + a_mat
        r1 = jnp.sum(jnp.where(lane == a1, rank, 0.0), axis=-1,
                     keepdims=True)
        r2 = jnp.sum(jnp.where(lane == a2, rank, 0.0), axis=-1,
                     keepdims=True)
        d1_ref[...] = a1 * CAP + r1.astype(jnp.int32)
        d2_ref[...] = a2 * CAP + r2.astype(jnp.int32)
        cnt_row = csum[L - 1:L, :]
        for k in range(N_EXPERTS):
            cnt_ref[k] = cnt_row[0, k].astype(jnp.int32)
        out_ref[...] = jnp.zeros_like(out_ref)

    cnt = cnt_ref[n]
    iota = jax.lax.broadcasted_iota(jnp.int32, (L, ROW_BLK), 1)

    def _block(j, carry):
        if True:
            slot = iota + (n * CAP + j * ROW_BLK)
            sel1 = slot == d1_ref[...]
            sel2 = slot == d2_ref[...]
            gt_b = (sel1 | sel2).astype(jnp.bfloat16)
            xs = jax.lax.dot_general(
                gt_b, x_ref[...].astype(jnp.bfloat16),
                (((0,), (0,)), ((), ())),
                preferred_element_type=jnp.float32).astype(jnp.bfloat16)
            h0 = jnp.dot(xs, w0_ref[0].astype(jnp.bfloat16),
                         preferred_element_type=jnp.float32)
            h1 = jnp.dot(xs, w1_ref[0].astype(jnp.bfloat16),
                         preferred_element_type=jnp.float32)
            m = ((h0 * jax.nn.sigmoid(h0)) * h1).astype(jnp.bfloat16)
            y = jnp.dot(m, wo_ref[0].astype(jnp.bfloat16),
                        preferred_element_type=jnp.float32)
            gt_w = (jnp.where(sel1, p1_ref[...], 0.0)
                    + jnp.where(sel2, p2_ref[...], 0.0)).astype(jnp.bfloat16)
            out_ref[...] += jnp.dot(gt_w, y.astype(jnp.bfloat16),
                                    preferred_element_type=jnp.float32)
        return carry

    jax.lax.fori_loop(0, (cnt + ROW_BLK - 1) // ROW_BLK, _block, 0)


@jax.jit
def _moe(inputs, gate_kernel, w0_kernel, w1_kernel, wo_kernel):
    x = inputs.reshape(L, E).astype(jnp.float32)
    out = pl.pallas_call(
        _moe_kernel,
        grid=(N_EXPERTS,),
        in_specs=[
            pl.BlockSpec((L, E), lambda n: (0, 0)),
            pl.BlockSpec((E, N_EXPERTS), lambda n: (0, 0)),
            pl.BlockSpec((1, E, MLP_DIM), lambda n: (n, 0, 0)),
            pl.BlockSpec((1, E, MLP_DIM), lambda n: (n, 0, 0)),
            pl.BlockSpec((1, MLP_DIM, E), lambda n: (n, 0, 0)),
        ],
        out_specs=pl.BlockSpec((L, E), lambda n: (0, 0)),
        out_shape=jax.ShapeDtypeStruct((L, E), jnp.float32),
        scratch_shapes=[
            pltpu.VMEM((L, 1), jnp.int32),
            pltpu.VMEM((L, 1), jnp.int32),
            pltpu.VMEM((L, 1), jnp.float32),
            pltpu.VMEM((L, 1), jnp.float32),
            pltpu.SMEM((N_EXPERTS,), jnp.int32),
        ],
        compiler_params=pltpu.CompilerParams(
            dimension_semantics=("arbitrary",),
            vmem_limit_bytes=100 * 1024 * 1024,
        ),
    )(x, gate_kernel, w0_kernel, w1_kernel, wo_kernel)
    return out.reshape(B, L, E)


def kernel(inputs, gate_kernel, w0_kernel, w1_kernel, wo_kernel):
    return _moe(inputs, gate_kernel, w0_kernel, w1_kernel, wo_kernel)


# speedup vs baseline: 1.2146x; 1.2146x over previous
import jax
import jax.numpy as jnp
from jax.experimental import pallas as pl
from jax.experimental.pallas import tpu as pltpu

B, L, E = 1, 2048, 768
N_EXPERTS = 8
TOP_K = 2
MLP_DIM = 2048

CAP = L
ROW_BLK = 256
BLKS_PER_E = CAP // ROW_BLK


def _moe_kernel(x_ref, gate_ref, w0_ref, w1_ref, wo_ref, out_ref,
                d1_ref, d2_ref, p1_ref, p2_ref, cnt_ref):
    n = pl.program_id(0)

    @pl.when(n == 0)
    def _route():
        logits = jnp.dot(x_ref[...], gate_ref[...],
                         preferred_element_type=jnp.float32)
        lane = jax.lax.broadcasted_iota(jnp.int32, (L, N_EXPERTS), 1)
        a1 = jnp.argmax(logits, axis=-1)[:, None]
        m1 = jnp.max(logits, axis=-1, keepdims=True)
        masked = jnp.where(lane == a1, -jnp.inf, logits)
        a2 = jnp.argmax(masked, axis=-1)[:, None]
        m2 = jnp.max(masked, axis=-1, keepdims=True)
        e2 = jnp.exp(m2 - m1)
        p1_ref[...] = 1.0 / (1.0 + e2)
        p2_ref[...] = e2 / (1.0 + e2)

        a_mat = ((lane == a1) | (lane == a2)).astype(jnp.float32)
        a_b = a_mat.astype(jnp.bfloat16)
        chunks = []
        for c in range(4):
            row = (jax.lax.broadcasted_iota(jnp.int32, (L // 4, L), 0)
                   + c * (L // 4))
            col = jax.lax.broadcasted_iota(jnp.int32, (L // 4, L), 1)
            lower_c = (row > col).astype(jnp.bfloat16)
            chunks.append(jnp.dot(lower_c, a_b,
                                  preferred_element_type=jnp.float32))
        rank = jnp.concatenate(chunks, axis=0)
        csum = rank + a_mat
        r1 = jnp.sum(jnp.where(lane == a1, rank, 0.0), axis=-1,
                     keepdims=True)
        r2 = jnp.sum(jnp.where(lane == a2, rank, 0.0), axis=-1,
                     keepdims=True)
        d1_ref[...] = a1 * CAP + r1.astype(jnp.int32)
        d2_ref[...] = a2 * CAP + r2.astype(jnp.int32)
        cnt_row = csum[L - 1:L, :]
        for k in range(N_EXPERTS):
            cnt_ref[k] = cnt_row[0, k].astype(jnp.int32)
        out_ref[...] = jnp.zeros_like(out_ref)

    cnt = cnt_ref[n]
    iota = jax.lax.broadcasted_iota(jnp.int32, (L, ROW_BLK), 1)

    def _block(j, carry):
        slot = iota + (n * CAP + j * ROW_BLK)
        sel1 = slot == d1_ref[...]
        sel2 = slot == d2_ref[...]
        gt_b = (sel1 | sel2).astype(jnp.bfloat16)
        xs = jax.lax.dot_general(
            gt_b, x_ref[...].astype(jnp.bfloat16),
            (((0,), (0,)), ((), ())),
            preferred_element_type=jnp.float32).astype(jnp.bfloat16)
        h0 = jnp.dot(xs, w0_ref[0].astype(jnp.bfloat16),
                     preferred_element_type=jnp.float32)
        h1 = jnp.dot(xs, w1_ref[0].astype(jnp.bfloat16),
                     preferred_element_type=jnp.float32)
        m = ((h0 * jax.nn.sigmoid(h0)) * h1).astype(jnp.bfloat16)
        y = jnp.dot(m, wo_ref[0].astype(jnp.bfloat16),
                    preferred_element_type=jnp.float32)
        gt_w = (jnp.where(sel1, p1_ref[...], 0.0)
                + jnp.where(sel2, p2_ref[...], 0.0)).astype(jnp.bfloat16)
        out_ref[...] += jnp.dot(gt_w, y.astype(jnp.bfloat16),
                                preferred_element_type=jnp.float32)
        return carry

    jax.lax.fori_loop(0, (cnt + ROW_BLK - 1) // ROW_BLK, _block, 0)


@jax.jit
def _moe(inputs, gate_kernel, w0_kernel, w1_kernel, wo_kernel):
    x = inputs.reshape(L, E).astype(jnp.float32)
    out = pl.pallas_call(
        _moe_kernel,
        grid=(N_EXPERTS,),
        in_specs=[
            pl.BlockSpec((L, E), lambda n: (0, 0)),
            pl.BlockSpec((E, N_EXPERTS), lambda n: (0, 0)),
            pl.BlockSpec((1, E, MLP_DIM), lambda n: (n, 0, 0)),
            pl.BlockSpec((1, E, MLP_DIM), lambda n: (n, 0, 0)),
            pl.BlockSpec((1, MLP_DIM, E), lambda n: (n, 0, 0)),
        ],
        out_specs=pl.BlockSpec((L, E), lambda n: (0, 0)),
        out_shape=jax.ShapeDtypeStruct((L, E), jnp.float32),
        scratch_shapes=[
            pltpu.VMEM((L, 1), jnp.int32),
            pltpu.VMEM((L, 1), jnp.int32),
            pltpu.VMEM((L, 1), jnp.float32),
            pltpu.VMEM((L, 1), jnp.float32),
            pltpu.SMEM((N_EXPERTS,), jnp.int32),
        ],
        compiler_params=pltpu.CompilerParams(
            dimension_semantics=("arbitrary",),
            vmem_limit_bytes=100 * 1024 * 1024,
        ),
    )(x, gate_kernel, w0_kernel, w1_kernel, wo_kernel)
    return out.reshape(B, L, E)


def kernel(inputs, gate_kernel, w0_kernel, w1_kernel, wo_kernel):
    return _moe(inputs, gate_kernel, w0_kernel, w1_kernel, wo_kernel)


# speedup vs baseline: 1.2265x; 1.0098x over previous
import jax
import jax.numpy as jnp
from jax.experimental import pallas as pl
from jax.experimental.pallas import tpu as pltpu

B, L, E = 1, 2048, 768
N_EXPERTS = 8
TOP_K = 2
MLP_DIM = 2048

CAP = L
ROW_BLK = 256
BLKS_PER_E = CAP // ROW_BLK


def _moe_kernel(x_ref, gate_ref, w0_ref, w1_ref, wo_ref, out_ref,
                d1_ref, d2_ref, p1_ref, p2_ref, cnt_ref, xb_ref):
    n = pl.program_id(0)

    @pl.when(n == 0)
    def _route():
        logits = jnp.dot(x_ref[...], gate_ref[...],
                         preferred_element_type=jnp.float32)
        lane = jax.lax.broadcasted_iota(jnp.int32, (L, N_EXPERTS), 1)
        a1 = jnp.argmax(logits, axis=-1)[:, None]
        m1 = jnp.max(logits, axis=-1, keepdims=True)
        masked = jnp.where(lane == a1, -jnp.inf, logits)
        a2 = jnp.argmax(masked, axis=-1)[:, None]
        m2 = jnp.max(masked, axis=-1, keepdims=True)
        e2 = jnp.exp(m2 - m1)
        p1_ref[...] = 1.0 / (1.0 + e2)
        p2_ref[...] = e2 / (1.0 + e2)

        a_mat = ((lane == a1) | (lane == a2)).astype(jnp.float32)
        a_b = a_mat.astype(jnp.bfloat16)
        chunks = []
        for c in range(4):
            row = (jax.lax.broadcasted_iota(jnp.int32, (L // 4, L), 0)
                   + c * (L // 4))
            col = jax.lax.broadcasted_iota(jnp.int32, (L // 4, L), 1)
            lower_c = (row > col).astype(jnp.bfloat16)
            chunks.append(jnp.dot(lower_c, a_b,
                                  preferred_element_type=jnp.float32))
        rank = jnp.concatenate(chunks, axis=0)
        csum = rank + a_mat
        r1 = jnp.sum(jnp.where(lane == a1, rank, 0.0), axis=-1,
                     keepdims=True)
        r2 = jnp.sum(jnp.where(lane == a2, rank, 0.0), axis=-1,
                     keepdims=True)
        d1_ref[...] = a1 * CAP + r1.astype(jnp.int32)
        d2_ref[...] = a2 * CAP + r2.astype(jnp.int32)
        cnt_row = csum[L - 1:L, :]
        for k in range(N_EXPERTS):
            cnt_ref[k] = cnt_row[0, k].astype(jnp.int32)
        out_ref[...] = jnp.zeros_like(out_ref)
        xb_ref[...] = x_ref[...].astype(jnp.bfloat16)

    cnt = cnt_ref[n]
    iota = jax.lax.broadcasted_iota(jnp.int32, (L, ROW_BLK), 1)

    def _block(j, carry):
        slot = iota + (n * CAP + j * ROW_BLK)
        sel1 = slot == d1_ref[...]
        sel2 = slot == d2_ref[...]
        gt_b = (sel1 | sel2).astype(jnp.bfloat16)
        xs = jax.lax.dot_general(
            gt_b, xb_ref[...], (((0,), (0,)), ((), ())),
            preferred_element_type=jnp.float32).astype(jnp.bfloat16)
        h0 = jnp.dot(xs, w0_ref[0].astype(jnp.bfloat16),
                     preferred_element_type=jnp.float32)
        h1 = jnp.dot(xs, w1_ref[0].astype(jnp.bfloat16),
                     preferred_element_type=jnp.float32)
        m = ((h0 * jax.nn.sigmoid(h0)) * h1).astype(jnp.bfloat16)
        y = jnp.dot(m, wo_ref[0].astype(jnp.bfloat16),
                    preferred_element_type=jnp.float32)
        gt_w = (jnp.where(sel1, p1_ref[...], 0.0)
                + jnp.where(sel2, p2_ref[...], 0.0)).astype(jnp.bfloat16)
        out_ref[...] += jnp.dot(gt_w, y.astype(jnp.bfloat16),
                                preferred_element_type=jnp.float32)
        return carry

    jax.lax.fori_loop(0, (cnt + ROW_BLK - 1) // ROW_BLK, _block, 0)


@jax.jit
def _moe(inputs, gate_kernel, w0_kernel, w1_kernel, wo_kernel):
    x = inputs.reshape(L, E).astype(jnp.float32)
    out = pl.pallas_call(
        _moe_kernel,
        grid=(N_EXPERTS,),
        in_specs=[
            pl.BlockSpec((L, E), lambda n: (0, 0)),
            pl.BlockSpec((E, N_EXPERTS), lambda n: (0, 0)),
            pl.BlockSpec((1, E, MLP_DIM), lambda n: (n, 0, 0)),
            pl.BlockSpec((1, E, MLP_DIM), lambda n: (n, 0, 0)),
            pl.BlockSpec((1, MLP_DIM, E), lambda n: (n, 0, 0)),
        ],
        out_specs=pl.BlockSpec((L, E), lambda n: (0, 0)),
        out_shape=jax.ShapeDtypeStruct((L, E), jnp.float32),
        scratch_shapes=[
            pltpu.VMEM((L, 1), jnp.int32),
            pltpu.VMEM((L, 1), jnp.int32),
            pltpu.VMEM((L, 1), jnp.float32),
            pltpu.VMEM((L, 1), jnp.float32),
            pltpu.SMEM((N_EXPERTS,), jnp.int32),
            pltpu.VMEM((L, E), jnp.bfloat16),
        ],
        compiler_params=pltpu.CompilerParams(
            dimension_semantics=("arbitrary",),
            vmem_limit_bytes=100 * 1024 * 1024,
        ),
    )(x, gate_kernel, w0_kernel, w1_kernel, wo_kernel)
    return out.reshape(B, L, E)


def kernel(inputs, gate_kernel, w0_kernel, w1_kernel, wo_kernel):
    return _moe(inputs, gate_kernel, w0_kernel, w1_kernel, wo_kernel)
